# baseline (speedup 1.0000x reference)
import functools

import jax
import jax.numpy as jnp
from jax import lax
from jax.experimental import pallas as pl
from jax.experimental.pallas import tpu as pltpu
from jax.experimental.pallas import tpu_sc as plsc

NNODE = 8000
NATTRI = 2000
N = NNODE + NATTRI
F = 128
E = 320000

NC = 2
NS = 16
CHUNK = 80
NCHUNK = 256
GRP = 16
NGRP = NCHUNK // GRP
EPT_PAD = NCHUNK * CHUNK
NPAD = 10240
ROWS_PER_TILE = NPAD // NS
ZROWS = CHUNK
NZ = ROWS_PER_TILE // ZROWS


def _prep_adj(adj_indices, adj_values):
    dst = adj_indices[0]
    src = adj_indices[1]
    pad = NS * EPT_PAD - E
    zi = jnp.zeros((pad,), jnp.int32)
    src = jnp.concatenate([src, zi]).reshape(NS, NCHUNK, CHUNK)
    dst = jnp.concatenate([dst, zi]).reshape(NS, NCHUNK, CHUNK)
    val = jnp.concatenate([adj_values, jnp.zeros((pad,), jnp.float32)])
    val = val.reshape(NS, NCHUNK, CHUNK)
    return src, dst, val


_RB = 1000


def _tc_body(node_ref, attri_ref, w1_ref, w2_ref, x1_ref, y_ref):
    i = pl.program_id(0)
    x = jnp.where(i < 8, node_ref[...], attri_ref[...])
    x1_ref[...] = x
    y_ref[0] = jnp.dot(x, w1_ref[...], preferred_element_type=jnp.float32)
    y_ref[1] = jnp.dot(x, w2_ref[...], preferred_element_type=jnp.float32)


def _tc_call(emb_node, emb_attri, W1, W2):
    return pl.pallas_call(
        _tc_body,
        grid=(N // _RB,),
        in_specs=[
            pl.BlockSpec((_RB, F), lambda i: (jnp.minimum(i, 7), 0)),
            pl.BlockSpec((_RB, F), lambda i: (jnp.maximum(i - 8, 0), 0)),
            pl.BlockSpec((F, F), lambda i: (0, 0)),
            pl.BlockSpec((F, F), lambda i: (0, 0)),
        ],
        out_specs=[
            pl.BlockSpec((_RB, F), lambda i: (i, 0)),
            pl.BlockSpec((2, _RB, F), lambda i: (0, i, 0)),
        ],
        out_shape=[
            jax.ShapeDtypeStruct((N, F), jnp.float32),
            jax.ShapeDtypeStruct((2, N, F), jnp.float32),
        ],
    )(emb_node, emb_attri, W1, W2)


_GATHER_DNUMS = lax.GatherDimensionNumbers(
    offset_dims=(), collapsed_slice_dims=(0,), start_index_map=(0,))


def _lane_broadcast(v16, r):
    idx = jnp.full((16, 1), r, dtype=jnp.int32)
    return lax.gather(v16, idx, _GATHER_DNUMS, (1,),
                      mode=lax.GatherScatterMode.PROMISE_IN_BOUNDS)

_sc_mesh = plsc.VectorSubcoreMesh(
    core_axis_name="c", subcore_axis_name="s", num_cores=NC, num_subcores=NS
)


@functools.partial(
    pl.kernel,
    out_type=jax.ShapeDtypeStruct((NC, NPAD, F), jnp.float32),
    mesh=_sc_mesh,
    scratch_types=[
        pltpu.VMEM((GRP, CHUNK), jnp.int32),
        pltpu.VMEM((GRP, CHUNK), jnp.int32),
        pltpu.VMEM((GRP, CHUNK), jnp.float32),
        pltpu.VMEM((CHUNK, F), jnp.float32),
        pltpu.VMEM((CHUNK, F), jnp.float32),
        pltpu.VMEM((CHUNK, F), jnp.float32),
        pltpu.VMEM((CHUNK, F), jnp.float32),
        pltpu.SemaphoreType.DMA,
        pltpu.SemaphoreType.DMA,
        pltpu.SemaphoreType.DMA,
        pltpu.SemaphoreType.DMA,
        pltpu.VMEM_SHARED((NPAD, F), jnp.float32),
    ],
)
def _sc_body(y_hbm, src_hbm, dst_hbm, val_hbm, out_hbm,
             src_v, dst_v, val_v, buf_a, buf_b, buf_s, buf_t,
             sem_ga, sem_gb, sem_ss, sem_st, acc):
    c = lax.axis_index("c")
    s = lax.axis_index("s")

    def dma_drain(buf, sem):
        pltpu.make_async_copy(y_hbm.at[c, pl.ds(0, CHUNK)], buf, sem).wait()

    zero = jnp.zeros((16,), jnp.float32)

    @plsc.parallel_loop(0, ZROWS, step=1, unroll=8)
    def _(r):
        for k in range(F // 16):
            buf_s[r, pl.ds(k * 16, 16)] = zero
    base = s * ROWS_PER_TILE
    for k in range(NZ):
        pltpu.sync_copy(buf_s, acc.at[pl.ds(base + k * ZROWS, ZROWS)])
    plsc.subcore_barrier()

    def scale_chunk(j, src_buf, dst_buf):
        @plsc.parallel_loop(0, CHUNK, step=1, unroll=8)
        def _(row):
            v16 = val_v[j, pl.ds((row // 16) * 16, 16)]
            bc = _lane_broadcast(v16, row % 16)
            for k in range(F // 16):
                dst_buf[row, pl.ds(k * 16, 16)] = (
                    src_buf[row, pl.ds(k * 16, 16)] * bc)

    def gather(j, buf, sem):
        pltpu.async_copy(y_hbm.at[c].at[src_v.at[j]], buf, sem)

    def scatter_add(j, buf, sem):
        pltpu.async_copy(buf, acc.at[dst_v.at[j]], sem, add=True)

    def group_fn(grp, carry0):
        off = pl.multiple_of(grp * GRP, 8)
        pltpu.sync_copy(src_hbm.at[c, s, pl.ds(off, GRP)], src_v)
        pltpu.sync_copy(dst_hbm.at[c, s, pl.ds(off, GRP)], dst_v)
        pltpu.sync_copy(val_hbm.at[c, s, pl.ds(off, GRP)], val_v)
        gather(0, buf_a, sem_ga)
        gather(1, buf_b, sem_gb)

        def pair(t2, carry):
            j0 = 2 * t2
            j1 = j0 + 1
            not_first = (grp > 0) | (t2 > 0)
            has_next = t2 < GRP // 2 - 1

            dma_drain(buf_a, sem_ga)
            pl.when(not_first)(lambda: dma_drain(buf_s, sem_ss))
            scale_chunk(j0, buf_a, buf_s)
            scatter_add(j0, buf_s, sem_ss)
            pl.when(has_next)(lambda: gather(j0 + 2, buf_a, sem_ga))

            dma_drain(buf_b, sem_gb)
            pl.when(not_first)(lambda: dma_drain(buf_t, sem_st))
            scale_chunk(j1, buf_b, buf_t)
            scatter_add(j1, buf_t, sem_st)
            pl.when(has_next)(lambda: gather(j1 + 2, buf_b, sem_gb))
            return carry

        lax.fori_loop(0, GRP // 2, pair, 0)
        return carry0

    lax.fori_loop(0, NGRP, group_fn, 0)

    dma_drain(buf_s, sem_ss)
    dma_drain(buf_t, sem_st)
    plsc.subcore_barrier()

    for k2 in range(NZ):
        pltpu.sync_copy(acc.at[pl.ds(base + k2 * ZROWS, ZROWS)], buf_s)

        @plsc.parallel_loop(0, ZROWS, step=1, unroll=8)
        def _(r):
            for k in range(F // 16):
                v = buf_s[r, pl.ds(k * 16, 16)]
                buf_s[r, pl.ds(k * 16, 16)] = jnp.maximum(v, 0.0)
        pltpu.sync_copy(buf_s, out_hbm.at[c, pl.ds(base + k2 * ZROWS, ZROWS)])


def kernel(adj_indices, adj_values, adj2_indices, adj2_values,
           emb_node, emb_attri, W1, W2):
    src1, dst1, val1 = _prep_adj(adj_indices, adj_values)
    src2, dst2, val2 = _prep_adj(adj2_indices, adj2_values)
    src = jnp.stack([src1, src2])
    dst = jnp.stack([dst1, dst2])
    val = jnp.stack([val1, val2])
    x1, y = _tc_call(emb_node, emb_attri, W1, W2)
    out = _sc_body(y, src, dst, val)
    return (x1, out[0, :N], out[1, :N])


# speedup vs baseline: 2.1748x; 2.1748x over previous
import functools

import jax
import jax.numpy as jnp
from jax import lax
from jax.experimental import pallas as pl
from jax.experimental.pallas import tpu as pltpu
from jax.experimental.pallas import tpu_sc as plsc

NNODE = 8000
NATTRI = 2000
N = NNODE + NATTRI
F = 128
E = 320000

NC = 2
NS = 16
CHUNK = 64
NCHUNK = 320
GRP = 32
NGRP = NCHUNK // GRP
EPT_PAD = NCHUNK * CHUNK
NPAD = 10240
ROWS_PER_TILE = NPAD // NS
ZROWS = CHUNK
NZ = ROWS_PER_TILE // ZROWS


def _prep_adj(adj_indices, adj_values):
    dst = adj_indices[0]
    src = adj_indices[1]
    pad = NS * EPT_PAD - E
    pi = jnp.arange(pad, dtype=jnp.int32) % N
    src = jnp.concatenate([src, pi]).reshape(NS, NCHUNK, CHUNK)
    dst = jnp.concatenate([dst, pi]).reshape(NS, NCHUNK, CHUNK)
    val = jnp.concatenate([adj_values, jnp.zeros((pad,), jnp.float32)])
    val = val.reshape(NS, NCHUNK, CHUNK)
    return src, dst, val


_RB = 1000


def _tc_body(node_ref, attri_ref, w1_ref, w2_ref, x1_ref, y_ref):
    i = pl.program_id(0)
    x = jnp.where(i < 8, node_ref[...], attri_ref[...])
    x1_ref[...] = x
    y_ref[0] = jnp.dot(x, w1_ref[...], preferred_element_type=jnp.float32)
    y_ref[1] = jnp.dot(x, w2_ref[...], preferred_element_type=jnp.float32)


def _tc_call(emb_node, emb_attri, W1, W2):
    return pl.pallas_call(
        _tc_body,
        grid=(N // _RB,),
        in_specs=[
            pl.BlockSpec((_RB, F), lambda i: (jnp.minimum(i, 7), 0)),
            pl.BlockSpec((_RB, F), lambda i: (jnp.maximum(i - 8, 0), 0)),
            pl.BlockSpec((F, F), lambda i: (0, 0)),
            pl.BlockSpec((F, F), lambda i: (0, 0)),
        ],
        out_specs=[
            pl.BlockSpec((_RB, F), lambda i: (i, 0)),
            pl.BlockSpec((2, _RB, F), lambda i: (0, i, 0)),
        ],
        out_shape=[
            jax.ShapeDtypeStruct((N, F), jnp.float32),
            jax.ShapeDtypeStruct((2, N, F), jnp.float32),
        ],
    )(emb_node, emb_attri, W1, W2)


_GATHER_DNUMS = lax.GatherDimensionNumbers(
    offset_dims=(), collapsed_slice_dims=(0,), start_index_map=(0,))


def _lane_broadcast(v16, r):
    idx = jnp.full((16, 1), r, dtype=jnp.int32)
    return lax.gather(v16, idx, _GATHER_DNUMS, (1,),
                      mode=lax.GatherScatterMode.PROMISE_IN_BOUNDS)

_sc_mesh = plsc.VectorSubcoreMesh(
    core_axis_name="c", subcore_axis_name="s", num_cores=NC, num_subcores=NS
)


@functools.partial(
    pl.kernel,
    out_type=jax.ShapeDtypeStruct((NC, NPAD, F), jnp.float32),
    mesh=_sc_mesh,
    scratch_types=[
        pltpu.VMEM((GRP, CHUNK), jnp.int32),
        pltpu.VMEM((GRP, CHUNK), jnp.int32),
        pltpu.VMEM((GRP, CHUNK), jnp.float32),
        pltpu.VMEM((CHUNK, F), jnp.float32),
        pltpu.VMEM((CHUNK, F), jnp.float32),
        pltpu.VMEM((CHUNK, F), jnp.float32),
        pltpu.VMEM((CHUNK, F), jnp.float32),
        pltpu.SemaphoreType.DMA,
        pltpu.SemaphoreType.DMA,
        pltpu.SemaphoreType.DMA,
        pltpu.SemaphoreType.DMA,
        pltpu.VMEM_SHARED((NPAD, F), jnp.float32),
    ],
)
def _sc_body(y_hbm, src_hbm, dst_hbm, val_hbm, out_hbm,
             src_v, dst_v, val_v, buf_a, buf_b, buf_s, buf_t,
             sem_ga, sem_gb, sem_ss, sem_st, acc):
    c = lax.axis_index("c")
    s = lax.axis_index("s")

    def dma_drain(buf, sem):
        pltpu.make_async_copy(y_hbm.at[c, pl.ds(0, CHUNK)], buf, sem).wait()

    zero = jnp.zeros((16,), jnp.float32)

    @plsc.parallel_loop(0, ZROWS, step=1, unroll=8)
    def _(r):
        for k in range(F // 16):
            buf_s[r, pl.ds(k * 16, 16)] = zero
    base = s * ROWS_PER_TILE
    for k in range(NZ):
        pltpu.sync_copy(buf_s, acc.at[pl.ds(base + k * ZROWS, ZROWS)])
    plsc.subcore_barrier()

    def scale_chunk(j, src_buf, dst_buf):
        @plsc.parallel_loop(0, CHUNK, step=1, unroll=8)
        def _(row):
            v16 = val_v[j, pl.ds((row // 16) * 16, 16)]
            bc = _lane_broadcast(v16, row % 16)
            for k in range(F // 16):
                dst_buf[row, pl.ds(k * 16, 16)] = (
                    src_buf[row, pl.ds(k * 16, 16)] * bc)

    def gather(j, buf, sem):
        pltpu.async_copy(y_hbm.at[c].at[src_v.at[j]], buf, sem)

    def scatter_add(j, buf, sem):
        pltpu.async_copy(buf, acc.at[dst_v.at[j]], sem, add=True)

    for grp in range(NGRP):
        off = grp * GRP
        pltpu.sync_copy(src_hbm.at[c, s, pl.ds(off, GRP)], src_v)
        pltpu.sync_copy(dst_hbm.at[c, s, pl.ds(off, GRP)], dst_v)
        pltpu.sync_copy(val_hbm.at[c, s, pl.ds(off, GRP)], val_v)
        gather(0, buf_a, sem_ga)
        gather(1, buf_b, sem_gb)

        first_grp = grp == 0

        def pair(t2, carry):
            j0 = 2 * t2
            j1 = j0 + 1
            not_first = (t2 > 0) if first_grp else (t2 >= 0)
            has_next = t2 < GRP // 2 - 1

            dma_drain(buf_a, sem_ga)
            pl.when(not_first)(lambda: dma_drain(buf_s, sem_ss))
            scale_chunk(j0, buf_a, buf_s)
            scatter_add(j0, buf_s, sem_ss)
            pl.when(has_next)(lambda: gather(j0 + 2, buf_a, sem_ga))

            dma_drain(buf_b, sem_gb)
            pl.when(not_first)(lambda: dma_drain(buf_t, sem_st))
            scale_chunk(j1, buf_b, buf_t)
            scatter_add(j1, buf_t, sem_st)
            pl.when(has_next)(lambda: gather(j1 + 2, buf_b, sem_gb))
            return carry

        lax.fori_loop(0, GRP // 2, pair, 0)

    dma_drain(buf_s, sem_ss)
    dma_drain(buf_t, sem_st)
    plsc.subcore_barrier()

    for k2 in range(NZ):
        pltpu.sync_copy(acc.at[pl.ds(base + k2 * ZROWS, ZROWS)], buf_s)

        @plsc.parallel_loop(0, ZROWS, step=1, unroll=8)
        def _(r):
            for k in range(F // 16):
                v = buf_s[r, pl.ds(k * 16, 16)]
                buf_s[r, pl.ds(k * 16, 16)] = jnp.maximum(v, 0.0)
        pltpu.sync_copy(buf_s, out_hbm.at[c, pl.ds(base + k2 * ZROWS, ZROWS)])


def kernel(adj_indices, adj_values, adj2_indices, adj2_values,
           emb_node, emb_attri, W1, W2):
    src1, dst1, val1 = _prep_adj(adj_indices, adj_values)
    src2, dst2, val2 = _prep_adj(adj2_indices, adj2_values)
    src = jnp.stack([src1, src2])
    dst = jnp.stack([dst1, dst2])
    val = jnp.stack([val1, val2])
    x1, y = _tc_call(emb_node, emb_attri, W1, W2)
    out = _sc_body(y, src, dst, val)
    return (x1, out[0, :N], out[1, :N])


# speedup vs baseline: 2.2278x; 1.0244x over previous
import functools

import jax
import jax.numpy as jnp
from jax import lax
from jax.experimental import pallas as pl
from jax.experimental.pallas import tpu as pltpu
from jax.experimental.pallas import tpu_sc as plsc

NNODE = 8000
NATTRI = 2000
N = NNODE + NATTRI
F = 128
E = 320000

NC = 2
NS = 16
CHUNK = 64
NCHUNK = 320
GRP = 32
NGRP = NCHUNK // GRP
EPT_PAD = NCHUNK * CHUNK
NPAD = 10240
ROWS_PER_TILE = NPAD // NS
ZROWS = CHUNK
NZ = ROWS_PER_TILE // ZROWS


def _prep_adj(adj_indices, adj_values):
    dst = adj_indices[0]
    src = adj_indices[1]
    pad = NS * EPT_PAD - E
    pi = jnp.arange(pad, dtype=jnp.int32) % N
    src = jnp.concatenate([src, pi]).reshape(NS, NCHUNK, CHUNK)
    dst = jnp.concatenate([dst, pi]).reshape(NS, NCHUNK, CHUNK)
    val = jnp.concatenate([adj_values, jnp.zeros((pad,), jnp.float32)])
    val = val.reshape(NS, NCHUNK, CHUNK)
    return src, dst, val


_RB = 1000


def _tc_body(node_ref, attri_ref, w1_ref, w2_ref, x1_ref, y_ref):
    i = pl.program_id(0)
    x = jnp.where(i < 8, node_ref[...], attri_ref[...])
    x1_ref[...] = x
    y_ref[0] = jnp.dot(x, w1_ref[...], preferred_element_type=jnp.float32)
    y_ref[1] = jnp.dot(x, w2_ref[...], preferred_element_type=jnp.float32)


def _tc_call(emb_node, emb_attri, W1, W2):
    return pl.pallas_call(
        _tc_body,
        grid=(N // _RB,),
        in_specs=[
            pl.BlockSpec((_RB, F), lambda i: (jnp.minimum(i, 7), 0)),
            pl.BlockSpec((_RB, F), lambda i: (jnp.maximum(i - 8, 0), 0)),
            pl.BlockSpec((F, F), lambda i: (0, 0)),
            pl.BlockSpec((F, F), lambda i: (0, 0)),
        ],
        out_specs=[
            pl.BlockSpec((_RB, F), lambda i: (i, 0)),
            pl.BlockSpec((2, _RB, F), lambda i: (0, i, 0)),
        ],
        out_shape=[
            jax.ShapeDtypeStruct((N, F), jnp.float32),
            jax.ShapeDtypeStruct((2, N, F), jnp.float32),
        ],
    )(emb_node, emb_attri, W1, W2)


_GATHER_DNUMS = lax.GatherDimensionNumbers(
    offset_dims=(), collapsed_slice_dims=(0,), start_index_map=(0,))


def _lane_broadcast(v16, r):
    idx = jnp.full((16, 1), r, dtype=jnp.int32)
    return lax.gather(v16, idx, _GATHER_DNUMS, (1,),
                      mode=lax.GatherScatterMode.PROMISE_IN_BOUNDS)

_sc_mesh = plsc.VectorSubcoreMesh(
    core_axis_name="c", subcore_axis_name="s", num_cores=NC, num_subcores=NS
)


@functools.partial(
    pl.kernel,
    out_type=[
        jax.ShapeDtypeStruct((N, F), jnp.float32),
        jax.ShapeDtypeStruct((N, F), jnp.float32),
    ],
    mesh=_sc_mesh,
    scratch_types=[
        pltpu.VMEM((GRP, CHUNK), jnp.int32),
        pltpu.VMEM((GRP, CHUNK), jnp.int32),
        pltpu.VMEM((GRP, CHUNK), jnp.float32),
        pltpu.VMEM((CHUNK, F), jnp.float32),
        pltpu.VMEM((CHUNK, F), jnp.float32),
        pltpu.VMEM((CHUNK, F), jnp.float32),
        pltpu.VMEM((CHUNK, F), jnp.float32),
        pltpu.SemaphoreType.DMA,
        pltpu.SemaphoreType.DMA,
        pltpu.SemaphoreType.DMA,
        pltpu.SemaphoreType.DMA,
        pltpu.VMEM_SHARED((NPAD, F), jnp.float32),
    ],
)
def _sc_body(y_hbm, src_hbm, dst_hbm, val_hbm, out2_hbm, out3_hbm,
             src_v, dst_v, val_v, buf_a, buf_b, buf_s, buf_t,
             sem_ga, sem_gb, sem_ss, sem_st, acc):
    c = lax.axis_index("c")
    s = lax.axis_index("s")

    def dma_drain(buf, sem):
        pltpu.make_async_copy(y_hbm.at[c, pl.ds(0, CHUNK)], buf, sem).wait()

    zero = jnp.zeros((16,), jnp.float32)

    @plsc.parallel_loop(0, ZROWS, step=1, unroll=8)
    def _(r):
        for k in range(F // 16):
            buf_s[r, pl.ds(k * 16, 16)] = zero
    base = s * ROWS_PER_TILE
    for k in range(NZ):
        pltpu.sync_copy(buf_s, acc.at[pl.ds(base + k * ZROWS, ZROWS)])
    plsc.subcore_barrier()

    def scale_chunk(j, src_buf, dst_buf):
        @plsc.parallel_loop(0, CHUNK, step=1, unroll=8)
        def _(row):
            v16 = val_v[j, pl.ds((row // 16) * 16, 16)]
            bc = _lane_broadcast(v16, row % 16)
            for k in range(F // 16):
                dst_buf[row, pl.ds(k * 16, 16)] = (
                    src_buf[row, pl.ds(k * 16, 16)] * bc)

    def gather(j, buf, sem):
        pltpu.async_copy(y_hbm.at[c].at[src_v.at[j]], buf, sem)

    def scatter_add(j, buf, sem):
        pltpu.async_copy(buf, acc.at[dst_v.at[j]], sem, add=True)

    for grp in range(NGRP):
        off = grp * GRP
        pltpu.sync_copy(src_hbm.at[c, s, pl.ds(off, GRP)], src_v)
        pltpu.sync_copy(dst_hbm.at[c, s, pl.ds(off, GRP)], dst_v)
        pltpu.sync_copy(val_hbm.at[c, s, pl.ds(off, GRP)], val_v)
        gather(0, buf_a, sem_ga)
        gather(1, buf_b, sem_gb)

        first_grp = grp == 0

        def pair(t2, carry):
            j0 = 2 * t2
            j1 = j0 + 1
            not_first = (t2 > 0) if first_grp else (t2 >= 0)
            has_next = t2 < GRP // 2 - 1

            dma_drain(buf_a, sem_ga)
            pl.when(not_first)(lambda: dma_drain(buf_s, sem_ss))
            scale_chunk(j0, buf_a, buf_s)
            scatter_add(j0, buf_s, sem_ss)
            pl.when(has_next)(lambda: gather(j0 + 2, buf_a, sem_ga))

            dma_drain(buf_b, sem_gb)
            pl.when(not_first)(lambda: dma_drain(buf_t, sem_st))
            scale_chunk(j1, buf_b, buf_t)
            scatter_add(j1, buf_t, sem_st)
            pl.when(has_next)(lambda: gather(j1 + 2, buf_b, sem_gb))
            return carry

        lax.fori_loop(0, GRP // 2, pair, 0)

    dma_drain(buf_s, sem_ss)
    dma_drain(buf_t, sem_st)
    plsc.subcore_barrier()

    def writeout(dst_ref):
        def wk(k2, carry):
            start = pl.multiple_of(base + k2 * ZROWS, 8)
            pltpu.sync_copy(acc.at[pl.ds(start, ZROWS)], buf_s)

            @plsc.parallel_loop(0, ZROWS, step=1, unroll=8)
            def _(r):
                for k in range(F // 16):
                    v = buf_s[r, pl.ds(k * 16, 16)]
                    buf_s[r, pl.ds(k * 16, 16)] = jnp.maximum(v, 0.0)

            pl.when(start + ZROWS <= N)(
                lambda: pltpu.sync_copy(buf_s, dst_ref.at[pl.ds(start, ZROWS)]))
            pl.when(start == N - 16)(
                lambda: pltpu.sync_copy(buf_s.at[pl.ds(0, 16)],
                                        dst_ref.at[pl.ds(N - 16, 16)]))
            return carry

        lax.fori_loop(0, NZ, wk, 0)

    pl.when(c == 0)(lambda: writeout(out2_hbm))
    pl.when(c == 1)(lambda: writeout(out3_hbm))


def kernel(adj_indices, adj_values, adj2_indices, adj2_values,
           emb_node, emb_attri, W1, W2):
    src1, dst1, val1 = _prep_adj(adj_indices, adj_values)
    src2, dst2, val2 = _prep_adj(adj2_indices, adj2_values)
    src = jnp.stack([src1, src2])
    dst = jnp.stack([dst1, dst2])
    val = jnp.stack([val1, val2])
    x1, y = _tc_call(emb_node, emb_attri, W1, W2)
    x2, x3 = _sc_body(y, src, dst, val)
    return (x1, x2, x3)


# speedup vs baseline: 2.2691x; 1.0185x over previous
import functools

import jax
import jax.numpy as jnp
from jax import lax
from jax.experimental import pallas as pl
from jax.experimental.pallas import tpu as pltpu
from jax.experimental.pallas import tpu_sc as plsc

NNODE = 8000
NATTRI = 2000
N = NNODE + NATTRI
F = 128
E = 320000

NC = 2
NS = 16
CHUNK = 64
NCHUNK = 320
GRP = 32
NGRP = NCHUNK // GRP
EPT_PAD = NCHUNK * CHUNK
NPAD = 10240
ROWS_PER_TILE = NPAD // NS
ZROWS = CHUNK
NZ = ROWS_PER_TILE // ZROWS


def _prep_adj(adj_indices, adj_values):
    dst = adj_indices[0]
    src = adj_indices[1]
    pad = NS * EPT_PAD - E
    pi = jnp.arange(pad, dtype=jnp.int32) % N
    src = jnp.concatenate([src, pi]).reshape(NS, NCHUNK, CHUNK)
    dst = jnp.concatenate([dst, pi]).reshape(NS, NCHUNK, CHUNK)
    val = jnp.concatenate([adj_values, jnp.zeros((pad,), jnp.float32)])
    val = val.reshape(NS, NCHUNK, CHUNK)
    return src, dst, val


_RB = 1000


def _tc_body(node_ref, attri_ref, w1_ref, w2_ref, x1_ref, y_ref):
    i = pl.program_id(0)
    x = jnp.where(i < 8, node_ref[...], attri_ref[...])
    x1_ref[...] = x
    y_ref[0] = jnp.dot(x, w1_ref[...], preferred_element_type=jnp.float32)
    y_ref[1] = jnp.dot(x, w2_ref[...], preferred_element_type=jnp.float32)


def _tc_call(emb_node, emb_attri, W1, W2):
    return pl.pallas_call(
        _tc_body,
        grid=(N // _RB,),
        in_specs=[
            pl.BlockSpec((_RB, F), lambda i: (jnp.minimum(i, 7), 0)),
            pl.BlockSpec((_RB, F), lambda i: (jnp.maximum(i - 8, 0), 0)),
            pl.BlockSpec((F, F), lambda i: (0, 0)),
            pl.BlockSpec((F, F), lambda i: (0, 0)),
        ],
        out_specs=[
            pl.BlockSpec((_RB, F), lambda i: (i, 0)),
            pl.BlockSpec((2, _RB, F), lambda i: (0, i, 0)),
        ],
        out_shape=[
            jax.ShapeDtypeStruct((N, F), jnp.float32),
            jax.ShapeDtypeStruct((2, N, F), jnp.float32),
        ],
    )(emb_node, emb_attri, W1, W2)


_GATHER_DNUMS = lax.GatherDimensionNumbers(
    offset_dims=(), collapsed_slice_dims=(0,), start_index_map=(0,))


def _lane_broadcast(v16, r):
    idx = jnp.full((16, 1), r, dtype=jnp.int32)
    return lax.gather(v16, idx, _GATHER_DNUMS, (1,),
                      mode=lax.GatherScatterMode.PROMISE_IN_BOUNDS)

_sc_mesh = plsc.VectorSubcoreMesh(
    core_axis_name="c", subcore_axis_name="s", num_cores=NC, num_subcores=NS
)


@functools.partial(
    pl.kernel,
    out_type=[
        jax.ShapeDtypeStruct((N, F), jnp.float32),
        jax.ShapeDtypeStruct((N, F), jnp.float32),
    ],
    mesh=_sc_mesh,
    scratch_types=[
        pltpu.VMEM((GRP, CHUNK), jnp.int32),
        pltpu.VMEM((GRP, CHUNK), jnp.int32),
        pltpu.VMEM((GRP, CHUNK), jnp.float32),
        pltpu.VMEM((CHUNK, F), jnp.float32),
        pltpu.VMEM((CHUNK, F), jnp.float32),
        pltpu.VMEM((CHUNK, F), jnp.float32),
        pltpu.VMEM((CHUNK, F), jnp.float32),
        pltpu.SemaphoreType.DMA,
        pltpu.SemaphoreType.DMA,
        pltpu.SemaphoreType.DMA,
        pltpu.SemaphoreType.DMA,
        pltpu.VMEM_SHARED((NPAD, F), jnp.float32),
    ],
)
def _sc_body(y_hbm, src_hbm, dst_hbm, val_hbm, out2_hbm, out3_hbm,
             src_v, dst_v, val_v, buf_a, buf_b, buf_s, buf_t,
             sem_ga, sem_gb, sem_ss, sem_st, acc):
    c = lax.axis_index("c")
    s = lax.axis_index("s")

    def dma_drain(buf, sem):
        pltpu.make_async_copy(y_hbm.at[c, pl.ds(0, CHUNK)], buf, sem).wait()

    zero = jnp.zeros((16,), jnp.float32)

    @plsc.parallel_loop(0, ZROWS, step=1, unroll=8)
    def _(r):
        for k in range(F // 16):
            buf_s[r, pl.ds(k * 16, 16)] = zero
    base = s * ROWS_PER_TILE
    for k in range(NZ):
        pltpu.sync_copy(buf_s, acc.at[pl.ds(base + k * ZROWS, ZROWS)])
    plsc.subcore_barrier()

    def scale_chunk(j, src_buf, dst_buf):
        @plsc.parallel_loop(0, CHUNK, step=1, unroll=16)
        def _(row):
            v16 = val_v[j, pl.ds((row // 16) * 16, 16)]
            bc = _lane_broadcast(v16, row % 16)
            for k in range(F // 16):
                dst_buf[row, pl.ds(k * 16, 16)] = (
                    src_buf[row, pl.ds(k * 16, 16)] * bc)

    def gather(j, buf, sem):
        pltpu.async_copy(y_hbm.at[c].at[src_v.at[j]], buf, sem)

    def scatter_add(j, buf, sem):
        pltpu.async_copy(buf, acc.at[dst_v.at[j]], sem, add=True)

    for grp in range(NGRP):
        off = grp * GRP
        pltpu.sync_copy(src_hbm.at[c, s, pl.ds(off, GRP)], src_v)
        pltpu.sync_copy(dst_hbm.at[c, s, pl.ds(off, GRP)], dst_v)
        pltpu.sync_copy(val_hbm.at[c, s, pl.ds(off, GRP)], val_v)
        gather(0, buf_a, sem_ga)
        gather(1, buf_b, sem_gb)

        first_grp = grp == 0

        def pair(t2, carry):
            j0 = 2 * t2
            j1 = j0 + 1
            not_first = (t2 > 0) if first_grp else (t2 >= 0)
            has_next = t2 < GRP // 2 - 1

            dma_drain(buf_a, sem_ga)
            pl.when(not_first)(lambda: dma_drain(buf_s, sem_ss))
            scale_chunk(j0, buf_a, buf_s)
            scatter_add(j0, buf_s, sem_ss)
            pl.when(has_next)(lambda: gather(j0 + 2, buf_a, sem_ga))

            dma_drain(buf_b, sem_gb)
            pl.when(not_first)(lambda: dma_drain(buf_t, sem_st))
            scale_chunk(j1, buf_b, buf_t)
            scatter_add(j1, buf_t, sem_st)
            pl.when(has_next)(lambda: gather(j1 + 2, buf_b, sem_gb))
            return carry

        lax.fori_loop(0, GRP // 2, pair, 0)

    dma_drain(buf_s, sem_ss)
    dma_drain(buf_t, sem_st)
    plsc.subcore_barrier()

    def writeout(dst_ref):
        def wk(k2, carry):
            start = pl.multiple_of(base + k2 * ZROWS, 8)
            pltpu.sync_copy(acc.at[pl.ds(start, ZROWS)], buf_s)

            @plsc.parallel_loop(0, ZROWS, step=1, unroll=8)
            def _(r):
                for k in range(F // 16):
                    v = buf_s[r, pl.ds(k * 16, 16)]
                    buf_s[r, pl.ds(k * 16, 16)] = jnp.maximum(v, 0.0)

            pl.when(start + ZROWS <= N)(
                lambda: pltpu.sync_copy(buf_s, dst_ref.at[pl.ds(start, ZROWS)]))
            pl.when(start == N - 16)(
                lambda: pltpu.sync_copy(buf_s.at[pl.ds(0, 16)],
                                        dst_ref.at[pl.ds(N - 16, 16)]))
            return carry

        lax.fori_loop(0, NZ, wk, 0)

    pl.when(c == 0)(lambda: writeout(out2_hbm))
    pl.when(c == 1)(lambda: writeout(out3_hbm))


def kernel(adj_indices, adj_values, adj2_indices, adj2_values,
           emb_node, emb_attri, W1, W2):
    src1, dst1, val1 = _prep_adj(adj_indices, adj_values)
    src2, dst2, val2 = _prep_adj(adj2_indices, adj2_values)
    src = jnp.stack([src1, src2])
    dst = jnp.stack([dst1, dst2])
    val = jnp.stack([val1, val2])
    x1, y = _tc_call(emb_node, emb_attri, W1, W2)
    x2, x3 = _sc_body(y, src, dst, val)
    return (x1, x2, x3)
